# Initial kernel scaffold; baseline (speedup 1.0000x reference)
import jax
import jax.numpy as jnp
from jax.experimental import pallas as pl


def kernel(char_ids, embed_table, conv_w, conv_b):
    raise NotImplementedError("write your pallas kernel here")


# speedup vs baseline: 3.1840x; 3.1840x over previous
import functools

import jax
import jax.numpy as jnp
from jax import lax
from jax.experimental import pallas as pl
from jax.experimental.pallas import tpu as pltpu
from jax.experimental.pallas import tpu_sc as plsc


def _tap_tables_tc(e_pad, conv_wt, conv_b2):
    def body(e_ref, w_ref, b_ref, out_ref):
        e = e_ref[...]
        for k in range(3):
            wk = w_ref[k]
            tk = lax.dot_general(
                e, wk,
                dimension_numbers=(((1,), (1,)), ((), ())),
                preferred_element_type=jnp.float32,
            )
            if k == 1:
                tk = tk + b_ref[...]
            out_ref[k] = tk
        return None

    return pl.pallas_call(
        body,
        out_shape=jax.ShapeDtypeStruct((3, 256, 32), jnp.float32),
    )(e_pad, conv_wt, conv_b2)


def _make_sc_kernel(n_tok):
    n_workers = 32
    tok_w = n_tok // n_workers
    groups = tok_w // 16
    ids_w = tok_w * 22
    out_w = tok_w * 32

    mesh = plsc.VectorSubcoreMesh(core_axis_name="c", subcore_axis_name="s")

    @functools.partial(
        pl.kernel,
        out_type=jax.ShapeDtypeStruct((n_tok * 32,), jnp.float32),
        mesh=mesh,
        scratch_types=[
            pltpu.VMEM((ids_w,), jnp.int32),
            pltpu.VMEM((3 * 256 * 32,), jnp.float32),
            pltpu.VMEM((out_w,), jnp.float32),
        ],
        compiler_params=pltpu.CompilerParams(needs_layout_passes=False),
    )
    def sc_main(ids_hbm, tab_hbm, out_hbm, ids_v, tab_v, out_v):
        wid = lax.axis_index("s") * 2 + lax.axis_index("c")
        pltpu.sync_copy(ids_hbm.at[pl.ds(wid * ids_w, ids_w)], ids_v)
        pltpu.sync_copy(tab_hbm, tab_v)

        iota = lax.iota(jnp.int32, 16)
        iota22 = iota * 22
        iota32 = iota * 32

        def group(g, carry):
            cbase = g * (16 * 22)
            obase = g * (16 * 32)
            c32 = []
            for e in range(22):
                c = plsc.load_gather(ids_v, [iota22 + (cbase + e)])
                c32.append(c * 32)
            for j in range(32):
                acc = jnp.full((16,), -jnp.inf, jnp.float32)
                for i in range(20):
                    f = 32 * i + j
                    w = f % 20
                    o = f // 20
                    v = (
                        plsc.load_gather(tab_v, [c32[w] + o])
                        + plsc.load_gather(tab_v, [c32[w + 1] + (8192 + o)])
                        + plsc.load_gather(tab_v, [c32[w + 2] + (16384 + o)])
                    )
                    acc = jnp.maximum(acc, v)
                plsc.store_scatter(out_v, [iota32 + (obase + j)], acc)
            return carry

        lax.fori_loop(0, groups, group, 0)
        pltpu.sync_copy(out_v, out_hbm.at[pl.ds(wid * out_w, out_w)])

    return sc_main


def kernel(char_ids, embed_table, conv_w, conv_b):
    b, s, w = char_ids.shape
    d = embed_table.shape[1]
    n_tok = b * s

    e_pad = jnp.pad(embed_table.astype(jnp.float32), ((0, 128), (0, 0)))
    conv_wt = conv_w.astype(jnp.float32).transpose(2, 0, 1)
    conv_b2 = conv_b.astype(jnp.float32).reshape(1, d)

    tables = _tap_tables_tc(e_pad, conv_wt, conv_b2)

    ids = char_ids.astype(jnp.int32).reshape(n_tok, w)
    ids_pad = jnp.pad(ids, ((0, 0), (1, 1)), constant_values=128)
    ids_flat = ids_pad.reshape(n_tok * 22)

    sc_main = _make_sc_kernel(n_tok)
    out_flat = sc_main(ids_flat, tables.reshape(3 * 256 * 32))
    return out_flat.reshape(b, s, d)
